# baseline (speedup 1.0000x reference)
import jax
import jax.numpy as jnp
from jax import lax
from jax.experimental import pallas as pl
from jax.experimental.pallas import tpu as pltpu
from jax.experimental.pallas import tpu_sc as plsc

HIDDEN_DIM = 128
MAX_DEPTH = 64
BATCH = 128
MAX_NODES = 256
VALUE_LEN = 32

NC, NS, L = 2, 16, 16
NW = NC * NS
TOTAL_NODES = BATCH * MAX_NODES
NODES_PER_W = TOTAL_NODES // NW
GROUP = 8
GROUPS_PER_W = NODES_PER_W // GROUP
VROWS = GROUP * VALUE_LEN
VIDX_PER_W = NODES_PER_W * VALUE_LEN // 128
COLS = HIDDEN_DIM // L
NBUF = 2


def _sc_body(nt_hbm, nv_hbm, dp_hbm, node_tab, val_tab, dep_tab, out_hbm,
             nidx, vidx, didx, vrows, nrows, drows, outv, sem0, sem1):
  wid = lax.axis_index("s") * NC + lax.axis_index("c")
  sems = (sem0, sem1)

  pltpu.sync_copy(nt_hbm.at[pl.ds(wid * NODES_PER_W, NODES_PER_W)], nidx)
  pltpu.sync_copy(dp_hbm.at[pl.ds(wid * NODES_PER_W, NODES_PER_W)], didx)
  pltpu.sync_copy(nv_hbm.at[pl.ds(wid * VIDX_PER_W, VIDX_PER_W)], vidx)

  def clamp_body(i, _):
    sl = pl.ds(i * L, L)
    didx[sl] = jnp.clip(didx[sl], 0, MAX_DEPTH - 1)
    return 0
  lax.fori_loop(0, NODES_PER_W // L, clamp_body, 0)

  def copies(g, b):
    cps = []
    for p in range(2):
      cps.append((val_tab.at[vidx.at[g * 2 + p]],
                  vrows.at[pl.ds((b * 2 + p) * 128, 128)]))
    cps.append((node_tab.at[nidx.at[pl.ds(g * GROUP, GROUP)]],
                nrows.at[pl.ds(b * GROUP, GROUP)]))
    cps.append((dep_tab.at[didx.at[pl.ds(g * GROUP, GROUP)]],
                drows.at[pl.ds(b * GROUP, GROUP)]))
    return cps

  def fire(g, b):
    for src, dst in copies(g, b):
      pltpu.async_copy(src, dst, sems[b])

  def drain(g, b):
    for src, dst in copies(g, b):
      pltpu.make_async_copy(src, dst, sems[b]).wait()

  fire(0, 0)
  fire(1, 1)

  def pair_body(t, _):
    for b in range(NBUF):
      g = t * NBUF + b
      drain(g, b)

      def node_body(i, _):
        vbase = (b * GROUP + i) * VALUE_LEN
        ri = b * GROUP + i
        for j in range(COLS):
          sl = pl.ds(j * L, L)
          acc = vrows[vbase, sl]
          for l in range(1, VALUE_LEN):
            acc = acc + vrows[vbase + l, sl]
          outv[ri, sl] = nrows[ri, sl] + drows[ri, sl] + acc * (1.0 / VALUE_LEN)
        return 0
      lax.fori_loop(0, GROUP, node_body, 0)

      base = (wid * GROUPS_PER_W + g) * GROUP
      pltpu.sync_copy(outv.at[pl.ds(b * GROUP, GROUP)],
                      out_hbm.at[pl.ds(base, GROUP)])

      @pl.when(g + NBUF < GROUPS_PER_W)
      def _():
        fire(g + NBUF, b)
    return 0

  lax.fori_loop(0, GROUPS_PER_W // NBUF, pair_body, 0)


@jax.jit
def _tree_embed(nt, nv, dp, node_tab, val_tab, dep_tab):
  mesh = plsc.VectorSubcoreMesh(
      core_axis_name="c", subcore_axis_name="s", num_cores=NC, num_subcores=NS)
  return pl.kernel(
      _sc_body,
      out_type=jax.ShapeDtypeStruct((TOTAL_NODES, HIDDEN_DIM), jnp.float32),
      mesh=mesh,
      scratch_types=[
          pltpu.VMEM((NODES_PER_W,), jnp.int32),
          pltpu.VMEM((VIDX_PER_W, 128), jnp.int32),
          pltpu.VMEM((NODES_PER_W,), jnp.int32),
          pltpu.VMEM((NBUF * VROWS, HIDDEN_DIM), jnp.float32),
          pltpu.VMEM((NBUF * GROUP, HIDDEN_DIM), jnp.float32),
          pltpu.VMEM((NBUF * GROUP, HIDDEN_DIM), jnp.float32),
          pltpu.VMEM((NBUF * GROUP, HIDDEN_DIM), jnp.float32),
          pltpu.SemaphoreType.DMA,
          pltpu.SemaphoreType.DMA,
      ],
  )(nt, nv, dp, node_tab, val_tab, dep_tab)


def kernel(node_types, node_values, depth, node_table, value_table, depth_table):
  nt = node_types.reshape(TOTAL_NODES).astype(jnp.int32)
  nv = node_values.reshape(TOTAL_NODES * VALUE_LEN // 128, 128).astype(jnp.int32)
  dp = depth.reshape(TOTAL_NODES).astype(jnp.int32)
  out = _tree_embed(nt, nv, dp, node_table, value_table, depth_table)
  return out.reshape(BATCH, MAX_NODES, HIDDEN_DIM)
